# baseline (speedup 1.0000x reference)
import functools

import jax
import jax.numpy as jnp
from jax import lax
from jax.experimental import pallas as pl
from jax.experimental.pallas import tpu as pltpu
from jax.experimental.pallas import tpu_sc as plsc

N_TILES = 32
N_CORES = 2
N_SUB = 16
LANES = 16


@functools.lru_cache(maxsize=None)
def _sc_deg_kernel(NT, CH, K, NP):
    per_tile = NP // N_SUB
    mesh = plsc.VectorSubcoreMesh(core_axis_name="c", subcore_axis_name="s")

    @functools.partial(
        pl.kernel,
        mesh=mesh,
        out_type=jax.ShapeDtypeStruct((N_CORES, NP), jnp.float32),
        scratch_types=[
            pltpu.VMEM((CH, K), jnp.int32),
            pltpu.VMEM((K,), jnp.float32),
            pltpu.VMEM((per_tile,), jnp.float32),
            pltpu.VMEM_SHARED((NP,), jnp.float32),
        ],
    )
    def k(dst_hbm, out_hbm, idx_v, ones_v, zbuf_v, acc):
        c = lax.axis_index("c")
        s = lax.axis_index("s")
        wid = c * N_SUB + s
        for i in range(K // LANES):
            ones_v[pl.ds(i * LANES, LANES)] = jnp.ones((LANES,), jnp.float32)
        for i in range(per_tile // LANES):
            zbuf_v[pl.ds(i * LANES, LANES)] = jnp.zeros((LANES,), jnp.float32)
        pltpu.sync_copy(zbuf_v, acc.at[pl.ds(s * per_tile, per_tile)])
        plsc.subcore_barrier()
        pltpu.sync_copy(dst_hbm.at[wid], idx_v)

        def body(j, carry):
            pltpu.sync_copy(ones_v, acc.at[idx_v.at[j]], add=True)
            return carry

        lax.fori_loop(0, CH, body, 0)
        plsc.subcore_barrier()
        pltpu.sync_copy(
            acc.at[pl.ds(s * per_tile, per_tile)],
            out_hbm.at[c, pl.ds(s * per_tile, per_tile)],
        )

    return k


@functools.lru_cache(maxsize=None)
def _sc_scatter_kernel(NPAD, D, NT, CH, K):
    rows_per_tile = NPAD // N_SUB
    ZR = 16
    n_zcopies = rows_per_tile // ZR
    mesh = plsc.VectorSubcoreMesh(core_axis_name="c", subcore_axis_name="s")

    assert CH % 4 == 0
    CH2 = CH // 2

    @functools.partial(
        pl.kernel,
        mesh=mesh,
        out_type=jax.ShapeDtypeStruct((N_CORES, NPAD, D), jnp.float32),
        scratch_types=[
            pltpu.VMEM((CH2, K), jnp.int32),
            pltpu.VMEM((CH2, K), jnp.int32),
            pltpu.VMEM((K, D), jnp.float32),
            pltpu.VMEM((K, D), jnp.float32),
            pltpu.VMEM((ZR, D), jnp.float32),
            pltpu.VMEM_SHARED((NPAD, D), jnp.float32),
            pltpu.SemaphoreType.DMA,
            pltpu.SemaphoreType.DMA,
        ],
    )
    def k(y_hbm, src_hbm, dst_hbm, out_hbm,
          isrc, idst, buf0, buf1, zbuf, acc, sem0, sem1):
        c = lax.axis_index("c")
        s = lax.axis_index("s")
        wid = c * N_SUB + s
        for r in range(ZR):
            for q in range(D // LANES):
                zbuf[r, pl.ds(q * LANES, LANES)] = jnp.zeros(
                    (LANES,), jnp.float32)
        for t in range(n_zcopies):
            pltpu.sync_copy(
                zbuf, acc.at[pl.ds(s * rows_per_tile + t * ZR, ZR)])
        plsc.subcore_barrier()

        for h in range(2):
            pltpu.sync_copy(src_hbm.at[wid, pl.ds(h * CH2, CH2)], isrc)
            pltpu.sync_copy(dst_hbm.at[wid, pl.ds(h * CH2, CH2)], idst)

            def body(j, carry):
                pltpu.async_copy(y_hbm.at[isrc.at[j]], buf0, sem0).wait()
                pltpu.sync_copy(buf0, acc.at[idst.at[j]], add=True)
                return carry

            lax.fori_loop(0, CH2, body, 0)

        plsc.subcore_barrier()
        pltpu.sync_copy(
            acc.at[pl.ds(s * rows_per_tile, rows_per_tile)],
            out_hbm.at[c, pl.ds(s * rows_per_tile, rows_per_tile)],
        )

    return k


def _tc_block(N):
    B = 2000
    assert N % B == 0
    return B, N // B


@functools.lru_cache(maxsize=None)
def _tc_y1_kernel(N, D):
    B, G = _tc_block(N)

    def body(x_ref, w_ref, dp_ref, o_ref):
        deg = dp_ref[0] + dp_ref[1] + 1.0
        dis = lax.rsqrt(deg)
        xw = jnp.dot(x_ref[...], w_ref[...],
                     preferred_element_type=jnp.float32,
                     precision=lax.Precision.HIGHEST)
        o_ref[...] = dis * xw

    return pl.pallas_call(
        body,
        grid=(G,),
        in_specs=[
            pl.BlockSpec((B, D), lambda i: (i, 0)),
            pl.BlockSpec((D, D), lambda i: (0, 0)),
            pl.BlockSpec((N_CORES, B, 1), lambda i: (0, i, 0)),
        ],
        out_specs=pl.BlockSpec((B, D), lambda i: (i, 0)),
        out_shape=jax.ShapeDtypeStruct((N, D), jnp.float32),
    )


@functools.lru_cache(maxsize=None)
def _tc_mid_kernel(N, D):
    B, G = _tc_block(N)

    def body(sp_ref, y_ref, dp_ref, b_ref, w_ref, o_ref):
        deg = dp_ref[0] + dp_ref[1] + 1.0
        dis = lax.rsqrt(deg)
        h = dis * (sp_ref[0] + sp_ref[1] + y_ref[...]) + b_ref[...]
        h = jnp.maximum(h, 0.0)
        hw = jnp.dot(h, w_ref[...],
                     preferred_element_type=jnp.float32,
                     precision=lax.Precision.HIGHEST)
        o_ref[...] = dis * hw

    return pl.pallas_call(
        body,
        grid=(G,),
        in_specs=[
            pl.BlockSpec((N_CORES, B, D), lambda i: (0, i, 0)),
            pl.BlockSpec((B, D), lambda i: (i, 0)),
            pl.BlockSpec((N_CORES, B, 1), lambda i: (0, i, 0)),
            pl.BlockSpec((1, D), lambda i: (0, 0)),
            pl.BlockSpec((D, D), lambda i: (0, 0)),
        ],
        out_specs=pl.BlockSpec((B, D), lambda i: (i, 0)),
        out_shape=jax.ShapeDtypeStruct((N, D), jnp.float32),
    )


@functools.lru_cache(maxsize=None)
def _tc_final_kernel(N, D):
    B, G = _tc_block(N)

    def body(sp_ref, y_ref, dp_ref, b_ref, o_ref):
        deg = dp_ref[0] + dp_ref[1] + 1.0
        dis = lax.rsqrt(deg)
        o_ref[...] = dis * (sp_ref[0] + sp_ref[1] + y_ref[...]) + b_ref[...]

    return pl.pallas_call(
        body,
        grid=(G,),
        in_specs=[
            pl.BlockSpec((N_CORES, B, D), lambda i: (0, i, 0)),
            pl.BlockSpec((B, D), lambda i: (i, 0)),
            pl.BlockSpec((N_CORES, B, 1), lambda i: (0, i, 0)),
            pl.BlockSpec((1, D), lambda i: (0, 0)),
        ],
        out_specs=pl.BlockSpec((B, D), lambda i: (i, 0)),
        out_shape=jax.ShapeDtypeStruct((N, D), jnp.float32),
    )


def kernel(x, edge_index, W1, b1, W2, b2):
    N, D = x.shape
    E = edge_index.shape[1]
    K = 128
    NP = 10240
    CH = -(-E // (N_TILES * K))
    CH += (-CH) % 4
    EPAD = N_TILES * CH * K
    pad = EPAD - E
    src_p = jnp.concatenate(
        [edge_index[0], jnp.zeros((pad,), jnp.int32)])
    dst_p = jnp.concatenate(
        [edge_index[1], N + (jnp.arange(pad, dtype=jnp.int32) % (NP - N))])
    src3 = src_p.reshape(N_TILES, CH, K)
    dst3 = dst_p.reshape(N_TILES, CH, K)

    degp = _sc_deg_kernel(N_TILES, CH, K, NP)(dst3)
    degp = degp[:, :N].reshape(N_CORES, N, 1)

    y1 = _tc_y1_kernel(N, D)(x, W1, degp)
    S1 = _sc_scatter_kernel(NP, D, N_TILES, CH, K)(y1, src3, dst3)[:, :N]
    y2 = _tc_mid_kernel(N, D)(S1, y1, degp, b1.reshape(1, D), W2)
    S2 = _sc_scatter_kernel(NP, D, N_TILES, CH, K)(y2, src3, dst3)[:, :N]
    out = _tc_final_kernel(N, D)(S2, y2, degp, b2.reshape(1, D))
    return out


# speedup vs baseline: 1.1948x; 1.1948x over previous
import functools

import jax
import jax.numpy as jnp
from jax import lax
from jax.experimental import pallas as pl
from jax.experimental.pallas import tpu as pltpu
from jax.experimental.pallas import tpu_sc as plsc

N_TILES = 32
N_CORES = 2
N_SUB = 16
LANES = 16


@functools.lru_cache(maxsize=None)
def _sc_deg_kernel(NT, CH, K, NP):
    per_tile = NP // N_SUB
    mesh = plsc.VectorSubcoreMesh(core_axis_name="c", subcore_axis_name="s")

    @functools.partial(
        pl.kernel,
        mesh=mesh,
        out_type=jax.ShapeDtypeStruct((N_CORES, NP), jnp.float32),
        scratch_types=[
            pltpu.VMEM((CH, K), jnp.int32),
            pltpu.VMEM((K,), jnp.float32),
            pltpu.VMEM((per_tile,), jnp.float32),
            pltpu.VMEM_SHARED((NP,), jnp.float32),
        ],
    )
    def k(dst_hbm, out_hbm, idx_v, ones_v, zbuf_v, acc):
        c = lax.axis_index("c")
        s = lax.axis_index("s")
        wid = c * N_SUB + s
        for i in range(K // LANES):
            ones_v[pl.ds(i * LANES, LANES)] = jnp.ones((LANES,), jnp.float32)
        for i in range(per_tile // LANES):
            zbuf_v[pl.ds(i * LANES, LANES)] = jnp.zeros((LANES,), jnp.float32)
        pltpu.sync_copy(zbuf_v, acc.at[pl.ds(s * per_tile, per_tile)])
        plsc.subcore_barrier()
        pltpu.sync_copy(dst_hbm.at[wid], idx_v)

        def body(j, carry):
            pltpu.sync_copy(ones_v, acc.at[idx_v.at[j]], add=True)
            return carry

        lax.fori_loop(0, CH, body, 0)
        plsc.subcore_barrier()
        pltpu.sync_copy(
            acc.at[pl.ds(s * per_tile, per_tile)],
            out_hbm.at[c, pl.ds(s * per_tile, per_tile)],
        )

    return k


@functools.lru_cache(maxsize=None)
def _sc_scatter_kernel(NPAD, D, NT, CH, K):
    rows_per_tile = NPAD // N_SUB
    ZR = 16
    n_zcopies = rows_per_tile // ZR
    mesh = plsc.VectorSubcoreMesh(core_axis_name="c", subcore_axis_name="s")

    assert CH % 4 == 0
    CH2 = CH // 2

    @functools.partial(
        pl.kernel,
        mesh=mesh,
        out_type=jax.ShapeDtypeStruct((N_CORES, NPAD, D), jnp.float32),
        scratch_types=[
            pltpu.VMEM((CH2, K), jnp.int32),
            pltpu.VMEM((CH2, K), jnp.int32),
            pltpu.VMEM((K, D), jnp.float32),
            pltpu.VMEM((K, D), jnp.float32),
            pltpu.VMEM((ZR, D), jnp.float32),
            pltpu.VMEM_SHARED((NPAD, D), jnp.float32),
            pltpu.SemaphoreType.DMA,
            pltpu.SemaphoreType.DMA,
        ],
    )
    def k(y_hbm, src_hbm, dst_hbm, out_hbm,
          isrc, idst, buf0, buf1, zbuf, acc, sem0, sem1):
        c = lax.axis_index("c")
        s = lax.axis_index("s")
        wid = c * N_SUB + s
        for r in range(ZR):
            for q in range(D // LANES):
                zbuf[r, pl.ds(q * LANES, LANES)] = jnp.zeros(
                    (LANES,), jnp.float32)
        for t in range(n_zcopies):
            pltpu.sync_copy(
                zbuf, acc.at[pl.ds(s * rows_per_tile + t * ZR, ZR)])
        plsc.subcore_barrier()

        for h in range(2):
            pltpu.sync_copy(src_hbm.at[wid, pl.ds(h * CH2, CH2)], isrc)
            pltpu.sync_copy(dst_hbm.at[wid, pl.ds(h * CH2, CH2)], idst)

            pltpu.async_copy(y_hbm.at[isrc.at[0]], buf0, sem0)

            def body(jj, carry):
                j0 = 2 * jj
                pltpu.async_copy(y_hbm.at[isrc.at[j0 + 1]], buf1, sem1)
                pltpu.make_async_copy(
                    y_hbm.at[isrc.at[j0]], buf0, sem0).wait()
                pltpu.sync_copy(buf0, acc.at[idst.at[j0]], add=True)
                pltpu.async_copy(y_hbm.at[isrc.at[j0 + 2]], buf0, sem0)
                pltpu.make_async_copy(
                    y_hbm.at[isrc.at[j0 + 1]], buf1, sem1).wait()
                pltpu.sync_copy(buf1, acc.at[idst.at[j0 + 1]], add=True)
                return carry

            lax.fori_loop(0, CH2 // 2 - 1, body, 0)
            pltpu.async_copy(y_hbm.at[isrc.at[CH2 - 1]], buf1, sem1)
            pltpu.make_async_copy(
                y_hbm.at[isrc.at[CH2 - 2]], buf0, sem0).wait()
            pltpu.sync_copy(buf0, acc.at[idst.at[CH2 - 2]], add=True)
            pltpu.make_async_copy(
                y_hbm.at[isrc.at[CH2 - 1]], buf1, sem1).wait()
            pltpu.sync_copy(buf1, acc.at[idst.at[CH2 - 1]], add=True)

        plsc.subcore_barrier()
        pltpu.sync_copy(
            acc.at[pl.ds(s * rows_per_tile, rows_per_tile)],
            out_hbm.at[c, pl.ds(s * rows_per_tile, rows_per_tile)],
        )

    return k


def _tc_block(N):
    B = 2000
    assert N % B == 0
    return B, N // B


@functools.lru_cache(maxsize=None)
def _tc_y1_kernel(N, D):
    B, G = _tc_block(N)

    def body(x_ref, w_ref, dp_ref, o_ref):
        deg = dp_ref[0] + dp_ref[1] + 1.0
        dis = lax.rsqrt(deg)
        xw = jnp.dot(x_ref[...], w_ref[...],
                     preferred_element_type=jnp.float32,
                     precision=lax.Precision.HIGHEST)
        o_ref[...] = dis * xw

    return pl.pallas_call(
        body,
        grid=(G,),
        in_specs=[
            pl.BlockSpec((B, D), lambda i: (i, 0)),
            pl.BlockSpec((D, D), lambda i: (0, 0)),
            pl.BlockSpec((N_CORES, B, 1), lambda i: (0, i, 0)),
        ],
        out_specs=pl.BlockSpec((B, D), lambda i: (i, 0)),
        out_shape=jax.ShapeDtypeStruct((N, D), jnp.float32),
    )


@functools.lru_cache(maxsize=None)
def _tc_mid_kernel(N, D):
    B, G = _tc_block(N)

    def body(sp_ref, y_ref, dp_ref, b_ref, w_ref, o_ref):
        deg = dp_ref[0] + dp_ref[1] + 1.0
        dis = lax.rsqrt(deg)
        h = dis * (sp_ref[0] + sp_ref[1] + y_ref[...]) + b_ref[...]
        h = jnp.maximum(h, 0.0)
        hw = jnp.dot(h, w_ref[...],
                     preferred_element_type=jnp.float32,
                     precision=lax.Precision.HIGHEST)
        o_ref[...] = dis * hw

    return pl.pallas_call(
        body,
        grid=(G,),
        in_specs=[
            pl.BlockSpec((N_CORES, B, D), lambda i: (0, i, 0)),
            pl.BlockSpec((B, D), lambda i: (i, 0)),
            pl.BlockSpec((N_CORES, B, 1), lambda i: (0, i, 0)),
            pl.BlockSpec((1, D), lambda i: (0, 0)),
            pl.BlockSpec((D, D), lambda i: (0, 0)),
        ],
        out_specs=pl.BlockSpec((B, D), lambda i: (i, 0)),
        out_shape=jax.ShapeDtypeStruct((N, D), jnp.float32),
    )


@functools.lru_cache(maxsize=None)
def _tc_final_kernel(N, D):
    B, G = _tc_block(N)

    def body(sp_ref, y_ref, dp_ref, b_ref, o_ref):
        deg = dp_ref[0] + dp_ref[1] + 1.0
        dis = lax.rsqrt(deg)
        o_ref[...] = dis * (sp_ref[0] + sp_ref[1] + y_ref[...]) + b_ref[...]

    return pl.pallas_call(
        body,
        grid=(G,),
        in_specs=[
            pl.BlockSpec((N_CORES, B, D), lambda i: (0, i, 0)),
            pl.BlockSpec((B, D), lambda i: (i, 0)),
            pl.BlockSpec((N_CORES, B, 1), lambda i: (0, i, 0)),
            pl.BlockSpec((1, D), lambda i: (0, 0)),
        ],
        out_specs=pl.BlockSpec((B, D), lambda i: (i, 0)),
        out_shape=jax.ShapeDtypeStruct((N, D), jnp.float32),
    )


def kernel(x, edge_index, W1, b1, W2, b2):
    N, D = x.shape
    E = edge_index.shape[1]
    K = 64
    NP = 10240
    CH = -(-E // (N_TILES * K))
    CH += (-CH) % 4
    EPAD = N_TILES * CH * K
    pad = EPAD - E
    src_p = jnp.concatenate(
        [edge_index[0], jnp.zeros((pad,), jnp.int32)])
    dst_p = jnp.concatenate(
        [edge_index[1], N + (jnp.arange(pad, dtype=jnp.int32) % (NP - N))])
    src3 = src_p.reshape(N_TILES, CH, K)
    dst3 = dst_p.reshape(N_TILES, CH, K)

    degp = _sc_deg_kernel(N_TILES, CH, K, NP)(dst3)
    degp = degp[:, :N].reshape(N_CORES, N, 1)

    y1 = _tc_y1_kernel(N, D)(x, W1, degp)
    S1 = _sc_scatter_kernel(NP, D, N_TILES, CH, K)(y1, src3, dst3)[:, :N]
    y2 = _tc_mid_kernel(N, D)(S1, y1, degp, b1.reshape(1, D), W2)
    S2 = _sc_scatter_kernel(NP, D, N_TILES, CH, K)(y2, src3, dst3)[:, :N]
    out = _tc_final_kernel(N, D)(S2, y2, degp, b2.reshape(1, D))
    return out


# speedup vs baseline: 3.0563x; 2.5580x over previous
import functools

import jax
import jax.numpy as jnp
from jax import lax
from jax.experimental import pallas as pl
from jax.experimental.pallas import tpu as pltpu
from jax.experimental.pallas import tpu_sc as plsc

N_TILES = 32
N_CORES = 2
N_SUB = 16
LANES = 16


@functools.lru_cache(maxsize=None)
def _sc_deg_kernel(NT, CH, K, NP):
    per_tile = NP // N_SUB
    mesh = plsc.VectorSubcoreMesh(core_axis_name="c", subcore_axis_name="s")

    @functools.partial(
        pl.kernel,
        mesh=mesh,
        out_type=jax.ShapeDtypeStruct((N_CORES, NP), jnp.float32),
        scratch_types=[
            pltpu.VMEM((CH, K), jnp.int32),
            pltpu.VMEM((K,), jnp.float32),
            pltpu.VMEM((per_tile,), jnp.float32),
            pltpu.VMEM_SHARED((NP,), jnp.float32),
        ],
    )
    def k(dst_hbm, out_hbm, idx_v, ones_v, zbuf_v, acc):
        c = lax.axis_index("c")
        s = lax.axis_index("s")
        wid = c * N_SUB + s
        for i in range(K // LANES):
            ones_v[pl.ds(i * LANES, LANES)] = jnp.ones((LANES,), jnp.float32)
        for i in range(per_tile // LANES):
            zbuf_v[pl.ds(i * LANES, LANES)] = jnp.zeros((LANES,), jnp.float32)
        pltpu.sync_copy(zbuf_v, acc.at[pl.ds(s * per_tile, per_tile)])
        plsc.subcore_barrier()
        pltpu.sync_copy(dst_hbm.at[wid], idx_v)

        def body(j, carry):
            pltpu.sync_copy(ones_v, acc.at[idx_v.at[j]], add=True)
            return carry

        lax.fori_loop(0, CH, body, 0)
        plsc.subcore_barrier()
        pltpu.sync_copy(
            acc.at[pl.ds(s * per_tile, per_tile)],
            out_hbm.at[c, pl.ds(s * per_tile, per_tile)],
        )

    return k


@functools.lru_cache(maxsize=None)
def _sc_scatter_kernel(NPAD, D, NT, CH, K):
    rows_per_tile = NPAD // N_SUB
    ZR = 16
    n_zcopies = rows_per_tile // ZR
    mesh = plsc.VectorSubcoreMesh(core_axis_name="c", subcore_axis_name="s")

    assert CH % 4 == 0
    CH2 = CH // 2

    @functools.partial(
        pl.kernel,
        mesh=mesh,
        out_type=jax.ShapeDtypeStruct((N_CORES, NPAD, D), jnp.float32),
        scratch_types=[
            pltpu.VMEM((CH2, K), jnp.int32),
            pltpu.VMEM((CH2, K), jnp.int32),
            pltpu.VMEM((K, D), jnp.float32),
            pltpu.VMEM((K, D), jnp.float32),
            pltpu.VMEM((ZR, D), jnp.float32),
            pltpu.VMEM_SHARED((NPAD, D), jnp.float32),
            pltpu.SemaphoreType.DMA,
            pltpu.SemaphoreType.DMA,
        ],
    )
    def k(y_hbm, src_hbm, dst_hbm, out_hbm,
          isrc, idst, buf0, buf1, zbuf, acc, sem0, sem1):
        c = lax.axis_index("c")
        s = lax.axis_index("s")
        wid = c * N_SUB + s
        for r in range(ZR):
            for q in range(D // LANES):
                zbuf[r, pl.ds(q * LANES, LANES)] = jnp.zeros(
                    (LANES,), jnp.float32)
        for t in range(n_zcopies):
            pltpu.sync_copy(
                zbuf, acc.at[pl.ds(s * rows_per_tile + t * ZR, ZR)])
        plsc.subcore_barrier()

        for h in range(2):
            pltpu.sync_copy(src_hbm.at[wid, pl.ds(h * CH2, CH2)], isrc)
            pltpu.sync_copy(dst_hbm.at[wid, pl.ds(h * CH2, CH2)], idst)

            pltpu.async_copy(y_hbm.at[isrc.at[0]], buf0, sem0)

            def body(jj, carry):
                j0 = 2 * jj
                pltpu.async_copy(y_hbm.at[isrc.at[j0 + 1]], buf1, sem1)
                pltpu.make_async_copy(
                    y_hbm.at[isrc.at[j0]], buf0, sem0).wait()
                pltpu.sync_copy(buf0, acc.at[idst.at[j0]], add=True)
                pltpu.async_copy(y_hbm.at[isrc.at[j0 + 2]], buf0, sem0)
                pltpu.make_async_copy(
                    y_hbm.at[isrc.at[j0 + 1]], buf1, sem1).wait()
                pltpu.sync_copy(buf1, acc.at[idst.at[j0 + 1]], add=True)
                return carry

            lax.fori_loop(0, CH2 // 2 - 1, body, 0)
            pltpu.async_copy(y_hbm.at[isrc.at[CH2 - 1]], buf1, sem1)
            pltpu.make_async_copy(
                y_hbm.at[isrc.at[CH2 - 2]], buf0, sem0).wait()
            pltpu.sync_copy(buf0, acc.at[idst.at[CH2 - 2]], add=True)
            pltpu.make_async_copy(
                y_hbm.at[isrc.at[CH2 - 1]], buf1, sem1).wait()
            pltpu.sync_copy(buf1, acc.at[idst.at[CH2 - 1]], add=True)

        plsc.subcore_barrier()
        pltpu.sync_copy(
            acc.at[pl.ds(s * rows_per_tile, rows_per_tile)],
            out_hbm.at[c, pl.ds(s * rows_per_tile, rows_per_tile)],
        )

    return k


def _tc_block(N):
    B = 2000
    assert N % B == 0
    return B, N // B


@functools.lru_cache(maxsize=None)
def _tc_y1_kernel(N, D):
    B, G = _tc_block(N)

    def body(x_ref, w_ref, dp_ref, o_ref):
        deg = dp_ref[0] + dp_ref[1] + 1.0
        dis = lax.rsqrt(deg)
        xw = jnp.dot(x_ref[...], w_ref[...],
                     preferred_element_type=jnp.float32,
                     precision=lax.Precision.HIGHEST)
        o_ref[...] = dis * xw

    return pl.pallas_call(
        body,
        grid=(G,),
        in_specs=[
            pl.BlockSpec((B, D), lambda i: (i, 0)),
            pl.BlockSpec((D, D), lambda i: (0, 0)),
            pl.BlockSpec((N_CORES, B, 1), lambda i: (0, i, 0)),
        ],
        out_specs=pl.BlockSpec((B, D), lambda i: (i, 0)),
        out_shape=jax.ShapeDtypeStruct((N, D), jnp.float32),
    )


@functools.lru_cache(maxsize=None)
def _tc_mid_kernel(N, D):
    B, G = _tc_block(N)

    def body(sp_ref, y_ref, dp_ref, b_ref, w_ref, o_ref):
        deg = dp_ref[0] + dp_ref[1] + 1.0
        dis = lax.rsqrt(deg)
        h = dis * (sp_ref[0] + sp_ref[1] + y_ref[...]) + b_ref[...]
        h = jnp.maximum(h, 0.0)
        hw = jnp.dot(h, w_ref[...],
                     preferred_element_type=jnp.float32,
                     precision=lax.Precision.HIGHEST)
        o_ref[...] = dis * hw

    return pl.pallas_call(
        body,
        grid=(G,),
        in_specs=[
            pl.BlockSpec((N_CORES, B, D), lambda i: (0, i, 0)),
            pl.BlockSpec((B, D), lambda i: (i, 0)),
            pl.BlockSpec((N_CORES, B, 1), lambda i: (0, i, 0)),
            pl.BlockSpec((1, D), lambda i: (0, 0)),
            pl.BlockSpec((D, D), lambda i: (0, 0)),
        ],
        out_specs=pl.BlockSpec((B, D), lambda i: (i, 0)),
        out_shape=jax.ShapeDtypeStruct((N, D), jnp.float32),
    )


@functools.lru_cache(maxsize=None)
def _tc_final_kernel(N, D):
    B, G = _tc_block(N)

    def body(sp_ref, y_ref, dp_ref, b_ref, o_ref):
        deg = dp_ref[0] + dp_ref[1] + 1.0
        dis = lax.rsqrt(deg)
        o_ref[...] = dis * (sp_ref[0] + sp_ref[1] + y_ref[...]) + b_ref[...]

    return pl.pallas_call(
        body,
        grid=(G,),
        in_specs=[
            pl.BlockSpec((N_CORES, B, D), lambda i: (0, i, 0)),
            pl.BlockSpec((B, D), lambda i: (i, 0)),
            pl.BlockSpec((N_CORES, B, 1), lambda i: (0, i, 0)),
            pl.BlockSpec((1, D), lambda i: (0, 0)),
        ],
        out_specs=pl.BlockSpec((B, D), lambda i: (i, 0)),
        out_shape=jax.ShapeDtypeStruct((N, D), jnp.float32),
    )


def kernel(x, edge_index, W1, b1, W2, b2):
    N, D = x.shape
    E = edge_index.shape[1]
    K = 64
    NP = 10240
    CH = -(-E // (N_TILES * K))
    CH += (-CH) % 4
    EPAD = N_TILES * CH * K
    pad = EPAD - E
    src_p = jnp.concatenate(
        [edge_index[0], jnp.arange(pad, dtype=jnp.int32) % N])
    dst_p = jnp.concatenate(
        [edge_index[1], N + (jnp.arange(pad, dtype=jnp.int32) % (NP - N))])
    src3 = src_p.reshape(N_TILES, CH, K)
    dst3 = dst_p.reshape(N_TILES, CH, K)

    degp = _sc_deg_kernel(N_TILES, CH, K, NP)(dst3)
    degp = degp[:, :N].reshape(N_CORES, N, 1)

    y1 = _tc_y1_kernel(N, D)(x, W1, degp)
    S1 = _sc_scatter_kernel(NP, D, N_TILES, CH, K)(y1, src3, dst3)[:, :N]
    y2 = _tc_mid_kernel(N, D)(S1, y1, degp, b1.reshape(1, D), W2)
    S2 = _sc_scatter_kernel(NP, D, N_TILES, CH, K)(y2, src3, dst3)[:, :N]
    out = _tc_final_kernel(N, D)(S2, y2, degp, b2.reshape(1, D))
    return out


# speedup vs baseline: 3.5588x; 1.1644x over previous
import functools

import jax
import jax.numpy as jnp
from jax import lax
from jax.experimental import pallas as pl
from jax.experimental.pallas import tpu as pltpu
from jax.experimental.pallas import tpu_sc as plsc

N_TILES = 32
N_CORES = 2
N_SUB = 16
LANES = 16


@functools.lru_cache(maxsize=None)
def _sc_deg_kernel(NT, CH, K, NP):
    per_tile = NP // N_SUB
    mesh = plsc.VectorSubcoreMesh(core_axis_name="c", subcore_axis_name="s")

    @functools.partial(
        pl.kernel,
        mesh=mesh,
        out_type=jax.ShapeDtypeStruct((N_CORES, NP), jnp.float32),
        scratch_types=[
            pltpu.VMEM((CH, K), jnp.int32),
            pltpu.VMEM((K,), jnp.float32),
            pltpu.VMEM((per_tile,), jnp.float32),
            pltpu.VMEM_SHARED((NP,), jnp.float32),
        ],
    )
    def k(dst_hbm, out_hbm, idx_v, ones_v, zbuf_v, acc):
        c = lax.axis_index("c")
        s = lax.axis_index("s")
        wid = c * N_SUB + s
        for i in range(K // LANES):
            ones_v[pl.ds(i * LANES, LANES)] = jnp.ones((LANES,), jnp.float32)
        for i in range(per_tile // LANES):
            zbuf_v[pl.ds(i * LANES, LANES)] = jnp.zeros((LANES,), jnp.float32)
        pltpu.sync_copy(zbuf_v, acc.at[pl.ds(s * per_tile, per_tile)])
        plsc.subcore_barrier()
        pltpu.sync_copy(dst_hbm.at[wid], idx_v)

        def body(j, carry):
            pltpu.sync_copy(ones_v, acc.at[idx_v.at[j]], add=True)
            return carry

        lax.fori_loop(0, CH, body, 0)
        plsc.subcore_barrier()
        pltpu.sync_copy(
            acc.at[pl.ds(s * per_tile, per_tile)],
            out_hbm.at[c, pl.ds(s * per_tile, per_tile)],
        )

    return k


@functools.lru_cache(maxsize=None)
def _sc_scatter_kernel(NPAD, D, NT, CH, K):
    rows_per_tile = NPAD // N_SUB
    ZR = 16
    n_zcopies = rows_per_tile // ZR
    mesh = plsc.VectorSubcoreMesh(core_axis_name="c", subcore_axis_name="s")

    assert CH % 4 == 0
    CH2 = CH // 2

    @functools.partial(
        pl.kernel,
        mesh=mesh,
        out_type=jax.ShapeDtypeStruct((N_CORES, NPAD, D), jnp.float32),
        scratch_types=[
            pltpu.VMEM((CH2, K), jnp.int32),
            pltpu.VMEM((CH2, K), jnp.int32),
            pltpu.VMEM((K, D), jnp.float32),
            pltpu.VMEM((K, D), jnp.float32),
            pltpu.VMEM((ZR, D), jnp.float32),
            pltpu.VMEM_SHARED((NPAD, D), jnp.float32),
            pltpu.SemaphoreType.DMA,
            pltpu.SemaphoreType.DMA,
        ],
    )
    def k(y_hbm, src_hbm, dst_hbm, out_hbm,
          isrc, idst, buf0, buf1, zbuf, acc, sem0, sem1):
        c = lax.axis_index("c")
        s = lax.axis_index("s")
        wid = c * N_SUB + s
        for r in range(ZR):
            for q in range(D // LANES):
                zbuf[r, pl.ds(q * LANES, LANES)] = jnp.zeros(
                    (LANES,), jnp.float32)
        for t in range(n_zcopies):
            pltpu.sync_copy(
                zbuf, acc.at[pl.ds(s * rows_per_tile + t * ZR, ZR)])
        plsc.subcore_barrier()

        for h in range(2):
            pltpu.sync_copy(src_hbm.at[wid, pl.ds(h * CH2, CH2)], isrc)
            pltpu.sync_copy(dst_hbm.at[wid, pl.ds(h * CH2, CH2)], idst)

            pltpu.async_copy(y_hbm.at[isrc.at[0]], buf0, sem0)

            def body(jj, carry):
                j0 = 2 * jj
                pltpu.async_copy(y_hbm.at[isrc.at[j0 + 1]], buf1, sem1)
                pltpu.make_async_copy(
                    y_hbm.at[isrc.at[j0]], buf0, sem0).wait()
                pltpu.sync_copy(buf0, acc.at[idst.at[j0]], add=True)
                pltpu.async_copy(y_hbm.at[isrc.at[j0 + 2]], buf0, sem0)
                pltpu.make_async_copy(
                    y_hbm.at[isrc.at[j0 + 1]], buf1, sem1).wait()
                pltpu.sync_copy(buf1, acc.at[idst.at[j0 + 1]], add=True)
                return carry

            lax.fori_loop(0, CH2 // 2 - 1, body, 0)
            pltpu.async_copy(y_hbm.at[isrc.at[CH2 - 1]], buf1, sem1)
            pltpu.make_async_copy(
                y_hbm.at[isrc.at[CH2 - 2]], buf0, sem0).wait()
            pltpu.sync_copy(buf0, acc.at[idst.at[CH2 - 2]], add=True)
            pltpu.make_async_copy(
                y_hbm.at[isrc.at[CH2 - 1]], buf1, sem1).wait()
            pltpu.sync_copy(buf1, acc.at[idst.at[CH2 - 1]], add=True)

        plsc.subcore_barrier()
        pltpu.sync_copy(
            acc.at[pl.ds(s * rows_per_tile, rows_per_tile)],
            out_hbm.at[c, pl.ds(s * rows_per_tile, rows_per_tile)],
        )

    return k


def _tc_block(N):
    B = 2000
    assert N % B == 0
    return B, N // B


@functools.lru_cache(maxsize=None)
def _tc_y1_kernel(N, D):
    B, G = _tc_block(N)

    def body(x_ref, w_ref, dp_ref, o_ref):
        deg = dp_ref[0] + dp_ref[1] + 1.0
        dis = lax.rsqrt(deg)
        xw = jnp.dot(x_ref[...], w_ref[...],
                     preferred_element_type=jnp.float32,
                     precision=lax.Precision.HIGHEST)
        o_ref[...] = dis * xw

    return pl.pallas_call(
        body,
        grid=(G,),
        in_specs=[
            pl.BlockSpec((B, D), lambda i: (i, 0)),
            pl.BlockSpec((D, D), lambda i: (0, 0)),
            pl.BlockSpec((N_CORES, B, 1), lambda i: (0, i, 0)),
        ],
        out_specs=pl.BlockSpec((B, D), lambda i: (i, 0)),
        out_shape=jax.ShapeDtypeStruct((N, D), jnp.float32),
    )


@functools.lru_cache(maxsize=None)
def _tc_mid_kernel(N, D):
    B, G = _tc_block(N)

    def body(sp_ref, y_ref, dp_ref, b_ref, w_ref, o_ref):
        deg = dp_ref[0] + dp_ref[1] + 1.0
        dis = lax.rsqrt(deg)
        h = dis * (sp_ref[0] + sp_ref[1] + y_ref[...]) + b_ref[...]
        h = jnp.maximum(h, 0.0)
        hw = jnp.dot(h, w_ref[...],
                     preferred_element_type=jnp.float32,
                     precision=lax.Precision.HIGHEST)
        o_ref[...] = dis * hw

    return pl.pallas_call(
        body,
        grid=(G,),
        in_specs=[
            pl.BlockSpec((N_CORES, B, D), lambda i: (0, i, 0)),
            pl.BlockSpec((B, D), lambda i: (i, 0)),
            pl.BlockSpec((N_CORES, B, 1), lambda i: (0, i, 0)),
            pl.BlockSpec((1, D), lambda i: (0, 0)),
            pl.BlockSpec((D, D), lambda i: (0, 0)),
        ],
        out_specs=pl.BlockSpec((B, D), lambda i: (i, 0)),
        out_shape=jax.ShapeDtypeStruct((N, D), jnp.float32),
    )


@functools.lru_cache(maxsize=None)
def _tc_final_kernel(N, D):
    B, G = _tc_block(N)

    def body(sp_ref, y_ref, dp_ref, b_ref, o_ref):
        deg = dp_ref[0] + dp_ref[1] + 1.0
        dis = lax.rsqrt(deg)
        o_ref[...] = dis * (sp_ref[0] + sp_ref[1] + y_ref[...]) + b_ref[...]

    return pl.pallas_call(
        body,
        grid=(G,),
        in_specs=[
            pl.BlockSpec((N_CORES, B, D), lambda i: (0, i, 0)),
            pl.BlockSpec((B, D), lambda i: (i, 0)),
            pl.BlockSpec((N_CORES, B, 1), lambda i: (0, i, 0)),
            pl.BlockSpec((1, D), lambda i: (0, 0)),
        ],
        out_specs=pl.BlockSpec((B, D), lambda i: (i, 0)),
        out_shape=jax.ShapeDtypeStruct((N, D), jnp.float32),
    )


def kernel(x, edge_index, W1, b1, W2, b2):
    N, D = x.shape
    E = edge_index.shape[1]
    K = 128
    NP = 10240
    CH = -(-E // (N_TILES * K))
    CH += (-CH) % 4
    EPAD = N_TILES * CH * K
    pad = EPAD - E
    src_p = jnp.concatenate(
        [edge_index[0], jnp.arange(pad, dtype=jnp.int32) % N])
    dst_p = jnp.concatenate(
        [edge_index[1], N + (jnp.arange(pad, dtype=jnp.int32) % (NP - N))])
    src3 = src_p.reshape(N_TILES, CH, K)
    dst3 = dst_p.reshape(N_TILES, CH, K)

    degp = _sc_deg_kernel(N_TILES, CH, K, NP)(dst3)
    degp = degp[:, :N].reshape(N_CORES, N, 1)

    y1 = _tc_y1_kernel(N, D)(x, W1, degp)
    S1 = _sc_scatter_kernel(NP, D, N_TILES, CH, K)(y1, src3, dst3)[:, :N]
    y2 = _tc_mid_kernel(N, D)(S1, y1, degp, b1.reshape(1, D), W2)
    S2 = _sc_scatter_kernel(NP, D, N_TILES, CH, K)(y2, src3, dst3)[:, :N]
    out = _tc_final_kernel(N, D)(S2, y2, degp, b2.reshape(1, D))
    return out
